# baseline (speedup 1.0000x reference)
import functools

import jax
import jax.numpy as jnp
from jax import lax
from jax.experimental import pallas as pl
from jax.experimental.pallas import tpu as pltpu
from jax.experimental.pallas import tpu_sc as plsc

NEG = 0.2
EPS = 1e-16
NC, NS, LANES = 2, 16, 16
NW = NC * NS
K = 128


def _dense_body(x_ref, wl_ref, attl_ref, attr_ref, w1_ref, att1_ref, att2_ref,
                xl_ref, al_ref, ar_ref, s1_ref, s2_ref):
    xb = x_ref[...]
    xl = jnp.dot(xb, wl_ref[...].T, preferred_element_type=jnp.float32)
    xl_ref[...] = xl
    x1 = jnp.dot(xb, w1_ref[...].T, preferred_element_type=jnp.float32)
    al_ref[...] = jnp.sum(xl * attl_ref[...], axis=1, keepdims=True)
    ar_ref[...] = jnp.sum(xl * attr_ref[...], axis=1, keepdims=True)
    s1_ref[...] = jnp.sum(x1 * att1_ref[...], axis=1, keepdims=True)
    s2_ref[...] = jnp.sum(x1 * att2_ref[...], axis=1, keepdims=True)


def _dense_call(x, Wl, attl, attr, W1, att1, att2, blk):
    n, f = x.shape
    c = Wl.shape[0]
    grid = n // blk
    full = lambda s: pl.BlockSpec(s, lambda i: (0, 0))
    return pl.pallas_call(
        _dense_body,
        grid=(grid,),
        in_specs=[
            pl.BlockSpec((blk, f), lambda i: (i, 0)),
            full((c, f)), full((1, c)), full((1, c)),
            full((c, f)), full((1, c)), full((1, c)),
        ],
        out_specs=[
            pl.BlockSpec((blk, c), lambda i: (i, 0)),
            pl.BlockSpec((blk, 1), lambda i: (i, 0)),
            pl.BlockSpec((blk, 1), lambda i: (i, 0)),
            pl.BlockSpec((blk, 1), lambda i: (i, 0)),
            pl.BlockSpec((blk, 1), lambda i: (i, 0)),
        ],
        out_shape=[
            jax.ShapeDtypeStruct((n, c), jnp.float32),
            jax.ShapeDtypeStruct((n, 1), jnp.float32),
            jax.ShapeDtypeStruct((n, 1), jnp.float32),
            jax.ShapeDtypeStruct((n, 1), jnp.float32),
            jax.ShapeDtypeStruct((n, 1), jnp.float32),
        ],
    )(x, Wl, attl, attr, W1, att1, att2)


def _mid_body(xl_ref, acc_ref, uc_ref, den_ref, b_ref,
              wl_ref, attl_ref, attr_ref, w1_ref, att1_ref, att2_ref,
              xl2_ref, al_ref, ar_ref, s1_ref, s2_ref):
    xl = xl_ref[...]
    total = acc_ref[0] + acc_ref[1] + xl * (uc_ref[0] + uc_ref[1])
    h = total / (den_ref[0] + den_ref[1] + EPS) + b_ref[...]
    h = jnp.maximum(h, 0.0)
    xl2 = jnp.dot(h, wl_ref[...].T, preferred_element_type=jnp.float32)
    xl2_ref[...] = xl2
    x1 = jnp.dot(h, w1_ref[...].T, preferred_element_type=jnp.float32)
    al_ref[...] = jnp.sum(xl2 * attl_ref[...], axis=1, keepdims=True)
    ar_ref[...] = jnp.sum(xl2 * attr_ref[...], axis=1, keepdims=True)
    s1_ref[...] = jnp.sum(x1 * att1_ref[...], axis=1, keepdims=True)
    s2_ref[...] = jnp.sum(x1 * att2_ref[...], axis=1, keepdims=True)


def _mid_call(xl, acc, uc, den, b, Wl, attl, attr, W1, att1, att2, blk):
    n, c1 = xl.shape
    c2 = Wl.shape[0]
    grid = n // blk
    full = lambda s: pl.BlockSpec(s, lambda i: tuple(0 for _ in s))
    return pl.pallas_call(
        _mid_body,
        grid=(grid,),
        in_specs=[
            pl.BlockSpec((blk, c1), lambda i: (i, 0)),
            pl.BlockSpec((2, blk, c1), lambda i: (0, i, 0)),
            pl.BlockSpec((2, blk, 1), lambda i: (0, i, 0)),
            pl.BlockSpec((2, blk, 1), lambda i: (0, i, 0)),
            full((1, c1)),
            full((c2, c1)), full((1, c2)), full((1, c2)),
            full((c2, c1)), full((1, c2)), full((1, c2)),
        ],
        out_specs=[
            pl.BlockSpec((blk, c2), lambda i: (i, 0)),
            pl.BlockSpec((blk, 1), lambda i: (i, 0)),
            pl.BlockSpec((blk, 1), lambda i: (i, 0)),
            pl.BlockSpec((blk, 1), lambda i: (i, 0)),
            pl.BlockSpec((blk, 1), lambda i: (i, 0)),
        ],
        out_shape=[
            jax.ShapeDtypeStruct((n, c2), jnp.float32),
            jax.ShapeDtypeStruct((n, 1), jnp.float32),
            jax.ShapeDtypeStruct((n, 1), jnp.float32),
            jax.ShapeDtypeStruct((n, 1), jnp.float32),
            jax.ShapeDtypeStruct((n, 1), jnp.float32),
        ],
    )(xl, acc, uc, den, b, Wl, attl, attr, W1, att1, att2)


def _final_body(xl_ref, acc_ref, uc_ref, den_ref, b_ref, out_ref):
    xl = xl_ref[...]
    total = acc_ref[0] + acc_ref[1] + xl * (uc_ref[0] + uc_ref[1])
    z = total / (den_ref[0] + den_ref[1] + EPS) + b_ref[...]
    m = jnp.max(z, axis=1, keepdims=True)
    e = jnp.exp(z - m)
    out_ref[...] = z - m - jnp.log(jnp.sum(e, axis=1, keepdims=True))


def _final_call(xl, acc, uc, den, b, blk):
    n, c = xl.shape
    grid = n // blk
    return pl.pallas_call(
        _final_body,
        grid=(grid,),
        in_specs=[
            pl.BlockSpec((blk, c), lambda i: (i, 0)),
            pl.BlockSpec((2, blk, c), lambda i: (0, i, 0)),
            pl.BlockSpec((2, blk, 1), lambda i: (0, i, 0)),
            pl.BlockSpec((2, blk, 1), lambda i: (0, i, 0)),
            pl.BlockSpec((1, c), lambda i: (0, 0)),
        ],
        out_specs=pl.BlockSpec((blk, c), lambda i: (i, 0)),
        out_shape=jax.ShapeDtypeStruct((n, c), jnp.float32),
    )(xl, acc, uc, den, b)


def _sc_layer_body(n, n_acc, c, eb, n_chunks, e_real, e_tot,
                   src_r, dst_r, al_r, ar_r, s1_r, s2_r, xl_r,
                   sig_r, den_r, uc_r, acc_r,
                   sidx0, sidx1, didx0, didx1, dsc0, dsc1,
                   al0, al1, ar0, ar1, s10, s11_, s20, s21_,
                   ex0, ex1, u0, u1, w0, w1, sg0, sg1,
                   rows0, rows1, zb_v,
                   den_sh, uc_sh, acc_sh,
                   sem_g0, sem_g1, sem_i0, sem_i1, sem_s0, sem_s1,
                   sem_a0, sem_a1):
    cid = lax.axis_index("c")
    sid = lax.axis_index("s")
    wid = sid * NC + cid
    base = wid * eb
    rpt = n_acc // NS
    row0 = pl.multiple_of(sid * rpt, K)

    zeros16 = jnp.zeros((LANES,), jnp.float32)

    @pl.loop(0, rpt // LANES)
    def _zs(i):
        zb_v[pl.ds(i * LANES, LANES)] = zeros16

    @pl.loop(0, K)
    def _zr(r):
        for g in range(c // LANES):
            rows0[r, pl.ds(g * LANES, LANES)] = zeros16

    pltpu.sync_copy(zb_v, den_sh.at[pl.ds(row0, rpt)])
    pltpu.sync_copy(zb_v, uc_sh.at[pl.ds(row0, rpt)])
    for k in range(rpt // K):
        pltpu.sync_copy(rows0, acc_sh.at[pl.ds(row0 + k * K, K)])

    plsc.subcore_barrier()

    sidx = (sidx0, sidx1)
    didx = (didx0, didx1)
    dsc = (dsc0, dsc1)
    alb = (al0, al1)
    arb = (ar0, ar1)
    s1b = (s10, s11_)
    s2b = (s20, s21_)
    exb = (ex0, ex1)
    ub = (u0, u1)
    wb = (w0, w1)
    sgb = (sg0, sg1)
    rows = (rows0, rows1)
    sem_g = (sem_g0, sem_g1)
    sem_i = (sem_i0, sem_i1)
    sem_s = (sem_s0, sem_s1)
    sem_a = (sem_a0, sem_a1)

    def drain_adds(b):
        pltpu.make_async_copy(exb[b], den_sh.at[pl.ds(0, K)], sem_a[b]).wait()
        pltpu.make_async_copy(ub[b], uc_sh.at[pl.ds(0, K)], sem_a[b]).wait()
        pltpu.make_async_copy(rows[b], acc_sh.at[pl.ds(0, K)], sem_a[b]).wait()

    def echunk(ci):
        cw = lax.rem(ci, n_chunks)
        return pl.multiple_of(base + cw * K, K)

    def fire_idx(ci, b):
        o = pl.multiple_of(jnp.minimum(echunk(ci), e_real - K), K)
        pltpu.async_copy(src_r.at[pl.ds(o, K)], sidx[b], sem_i[b])
        pltpu.async_copy(dst_r.at[pl.ds(o, K)], didx[b], sem_i[b])

    def fixup(ci, b):
        o = echunk(ci)
        for j in range(K // LANES):
            sl = pl.ds(j * LANES, LANES)
            gid = o + j * LANES + lax.iota(jnp.int32, LANES)
            synth = gid >= e_real
            idxs = jnp.minimum(gid - e_real, n - 1)
            sidx[b][sl] = jnp.where(synth, idxs, sidx[b][sl])
            didx[b][sl] = jnp.where(synth, idxs, didx[b][sl])

    def wait_idx(b):
        pltpu.make_async_copy(src_r.at[pl.ds(0, K)], sidx[b], sem_i[b]).wait()
        pltpu.make_async_copy(dst_r.at[pl.ds(0, K)], didx[b], sem_i[b]).wait()

    def fire_gathers(b):
        pltpu.async_copy(al_r.at[sidx[b]], alb[b], sem_g[b])
        pltpu.async_copy(ar_r.at[didx[b]], arb[b], sem_g[b])
        pltpu.async_copy(s1_r.at[sidx[b]], s1b[b], sem_g[b])
        pltpu.async_copy(s2_r.at[didx[b]], s2b[b], sem_g[b])
        pltpu.async_copy(xl_r.at[sidx[b]], rows[b], sem_g[b])

    def wait_gathers(b):
        pltpu.make_async_copy(al_r.at[sidx[b]], alb[b], sem_g[b]).wait()
        pltpu.make_async_copy(ar_r.at[didx[b]], arb[b], sem_g[b]).wait()
        pltpu.make_async_copy(s1_r.at[sidx[b]], s1b[b], sem_g[b]).wait()
        pltpu.make_async_copy(s2_r.at[didx[b]], s2b[b], sem_g[b]).wait()
        pltpu.make_async_copy(xl_r.at[sidx[b]], rows[b], sem_g[b]).wait()

    def compute(ci, b):
        o = echunk(ci)
        for j in range(K // LANES):
            sl = pl.ds(j * LANES, LANES)
            a = alb[b][sl] + arb[b][sl]
            a = jnp.maximum(a, a * NEG)
            ex = jnp.exp(a)
            t = s1b[b][sl] + s2b[b][sl]
            t = jnp.maximum(t, t * NEG)
            sg = 1.0 / (1.0 + jnp.exp(-t))
            gid = o + j * LANES + lax.iota(jnp.int32, LANES)
            ex = ex * (gid < e_tot).astype(jnp.float32)
            w16 = ex * sg
            sgb[b][sl] = sg
            exb[b][sl] = ex
            ub[b][sl] = ex - w16
            wb[b][sl] = w16
            dsc[b][sl] = didx[b][sl]

    def scale(b):
        @pl.loop(0, K, unroll=4)
        def _scale(e):
            bw = plsc.load_gather(wb[b], [jnp.broadcast_to(e, (LANES,))])
            for g in range(c // LANES):
                rows[b][e, pl.ds(g * LANES, LANES)] = (
                    rows[b][e, pl.ds(g * LANES, LANES)] * bw)

    ob = pl.multiple_of(jnp.minimum(base, e_real - K), K)
    pltpu.sync_copy(src_r.at[pl.ds(ob, K)], sidx[0])
    pltpu.sync_copy(dst_r.at[pl.ds(ob, K)], didx[0])
    fixup(0, 0)
    fire_gathers(0)
    fire_idx(1, 1)

    @pl.loop(0, n_chunks, step=2)
    def _chunk(co):
        for b in range(2):
            ci = co + b
            nb = 1 - b
            wait_idx(nb)
            fixup(ci + 1, nb)

            @pl.when(ci > 0)
            def _():
                pltpu.make_async_copy(
                    sgb[nb], sig_r.at[pl.ds(0, K)], sem_s[nb]).wait()
                drain_adds(nb)
            fire_gathers(nb)
            wait_gathers(b)
            compute(ci, b)
            o = echunk(ci)
            pltpu.async_copy(sgb[b], sig_r.at[pl.ds(o, K)], sem_s[b])
            pltpu.async_copy(exb[b], den_sh.at[dsc[b]], sem_a[b], add=True)
            pltpu.async_copy(ub[b], uc_sh.at[dsc[b]], sem_a[b], add=True)
            fire_idx(ci + 2, b)
            scale(b)
            pltpu.async_copy(rows[b], acc_sh.at[dsc[b]], sem_a[b], add=True)

    wait_idx(1)
    wait_gathers(0)
    pltpu.make_async_copy(sgb[1], sig_r.at[pl.ds(0, K)], sem_s[1]).wait()
    drain_adds(1)

    plsc.subcore_barrier()

    @pl.when(sid == 0)
    def _out_scal():
        cb0 = pl.multiple_of(cid * n_acc, K)
        pltpu.sync_copy(den_sh, den_r.at[pl.ds(cb0, n_acc)])
        pltpu.sync_copy(uc_sh, uc_r.at[pl.ds(cb0, n_acc)])

    ob = pl.multiple_of(cid * n_acc + row0, K)
    for k in range(rpt // K):
        pltpu.sync_copy(acc_sh.at[pl.ds(row0 + k * K, K)],
                        acc_r.at[pl.ds(ob + k * K, K)])


def _sc_layer(src, dst, al, ar, s1, s2, xl, e_tot, ep):
    n, c = xl.shape
    n_acc = ((n + NS * K - 1) // (NS * K)) * (NS * K)
    eb = ep // NW
    n_chunks = eb // K
    e_real = src.shape[0]
    body = functools.partial(_sc_layer_body, n, n_acc, c, eb, n_chunks,
                             e_real, e_tot)
    mesh = plsc.VectorSubcoreMesh(core_axis_name="c", subcore_axis_name="s",
                                  num_cores=NC, num_subcores=NS)
    f32, i32 = jnp.float32, jnp.int32
    fn = pl.kernel(
        body,
        out_type=[
            jax.ShapeDtypeStruct((ep,), f32),
            jax.ShapeDtypeStruct((NC * n_acc,), f32),
            jax.ShapeDtypeStruct((NC * n_acc,), f32),
            jax.ShapeDtypeStruct((NC * n_acc, c), f32),
        ],
        mesh=mesh,
        scratch_types=(
            [pltpu.VMEM((K,), i32) for _ in range(6)]
            + [pltpu.VMEM((K,), f32) for _ in range(16)]
            + [pltpu.VMEM((K, c), f32) for _ in range(2)]
            + [pltpu.VMEM((n_acc // NS,), f32)]
            + [pltpu.VMEM_SHARED((n_acc,), f32),
               pltpu.VMEM_SHARED((n_acc,), f32),
               pltpu.VMEM_SHARED((n_acc, c), f32)]
            + [pltpu.SemaphoreType.DMA for _ in range(8)]
        ),
        compiler_params=pltpu.CompilerParams(needs_layout_passes=False,
                                             use_tc_tiling_on_sc=False),
    )
    sig, den, uc, acc = fn(src, dst, al, ar, s1, s2, xl)
    den = den.reshape(NC, n_acc, 1)
    uc = uc.reshape(NC, n_acc, 1)
    acc = acc.reshape(NC, n_acc, c)
    return sig, den, uc, acc


def kernel(x, edge_index, Wl1, attl1, attr1, W11, att11, att12, b1,
           Wl2, attl2, attr2, W12, att21, att22, b2):
    n, _ = x.shape
    e = edge_index.shape[1]
    e_tot = e + n
    ep = ((e_tot + NW * 2 * K - 1) // (NW * 2 * K)) * (NW * 2 * K)
    src = edge_index[0]
    dst = edge_index[1]

    blk = 400

    xl1, al1, ar1, s11, s21 = _dense_call(
        x, Wl1, attl1.reshape(1, -1), attr1.reshape(1, -1),
        W11, att11.reshape(1, -1), att12.reshape(1, -1), blk)

    sig1p, den1, uc1, acc1 = _sc_layer(
        src, dst, al1.reshape(-1), ar1.reshape(-1),
        s11.reshape(-1), s21.reshape(-1), xl1, e_tot, ep)

    xl2, al2, ar2, s12, s22 = _mid_call(
        xl1, acc1, uc1, den1,
        b1.reshape(1, -1), Wl2, attl2.reshape(1, -1), attr2.reshape(1, -1),
        W12, att21.reshape(1, -1), att22.reshape(1, -1), blk)

    sig2p, den2, uc2, acc2 = _sc_layer(
        src, dst, al2.reshape(-1), ar2.reshape(-1),
        s12.reshape(-1), s22.reshape(-1), xl2, e_tot, ep)

    out = _final_call(xl2, acc2, uc2, den2, b2.reshape(1, -1), blk)

    return out, sig1p[:e_tot], sig2p[:e_tot]


# speedup vs baseline: 1.5821x; 1.5821x over previous
import functools

import jax
import jax.numpy as jnp
from jax import lax
from jax.experimental import pallas as pl
from jax.experimental.pallas import tpu as pltpu
from jax.experimental.pallas import tpu_sc as plsc

NEG = 0.2
EPS = 1e-16
NC, NS, LANES = 2, 16, 16
NW = NC * NS
K = 128


def _dense_body(x_ref, wl_ref, attl_ref, attr_ref, w1_ref, att1_ref, att2_ref,
                xl_ref, al_ref, ar_ref, s1_ref, s2_ref):
    xb = x_ref[...]
    xl = jnp.dot(xb, wl_ref[...].T, preferred_element_type=jnp.float32)
    xl_ref[...] = xl
    x1 = jnp.dot(xb, w1_ref[...].T, preferred_element_type=jnp.float32)
    al_ref[...] = jnp.sum(xl * attl_ref[...], axis=1, keepdims=True)
    ar_ref[...] = jnp.sum(xl * attr_ref[...], axis=1, keepdims=True)
    s1_ref[...] = jnp.sum(x1 * att1_ref[...], axis=1, keepdims=True)
    s2_ref[...] = jnp.sum(x1 * att2_ref[...], axis=1, keepdims=True)


def _dense_call(x, Wl, attl, attr, W1, att1, att2, blk):
    n, f = x.shape
    c = Wl.shape[0]
    grid = n // blk
    full = lambda s: pl.BlockSpec(s, lambda i: (0, 0))
    return pl.pallas_call(
        _dense_body,
        grid=(grid,),
        in_specs=[
            pl.BlockSpec((blk, f), lambda i: (i, 0)),
            full((c, f)), full((1, c)), full((1, c)),
            full((c, f)), full((1, c)), full((1, c)),
        ],
        out_specs=[
            pl.BlockSpec((blk, c), lambda i: (i, 0)),
            pl.BlockSpec((blk, 1), lambda i: (i, 0)),
            pl.BlockSpec((blk, 1), lambda i: (i, 0)),
            pl.BlockSpec((blk, 1), lambda i: (i, 0)),
            pl.BlockSpec((blk, 1), lambda i: (i, 0)),
        ],
        out_shape=[
            jax.ShapeDtypeStruct((n, c), jnp.float32),
            jax.ShapeDtypeStruct((n, 1), jnp.float32),
            jax.ShapeDtypeStruct((n, 1), jnp.float32),
            jax.ShapeDtypeStruct((n, 1), jnp.float32),
            jax.ShapeDtypeStruct((n, 1), jnp.float32),
        ],
    )(x, Wl, attl, attr, W1, att1, att2)


def _mid_body(xl_ref, acc_ref, uc_ref, den_ref, b_ref,
              wl_ref, attl_ref, attr_ref, w1_ref, att1_ref, att2_ref,
              xl2_ref, al_ref, ar_ref, s1_ref, s2_ref):
    xl = xl_ref[...]
    total = acc_ref[0] + acc_ref[1] + xl * (uc_ref[0] + uc_ref[1])
    h = total / (den_ref[0] + den_ref[1] + EPS) + b_ref[...]
    h = jnp.maximum(h, 0.0)
    xl2 = jnp.dot(h, wl_ref[...].T, preferred_element_type=jnp.float32)
    xl2_ref[...] = xl2
    x1 = jnp.dot(h, w1_ref[...].T, preferred_element_type=jnp.float32)
    al_ref[...] = jnp.sum(xl2 * attl_ref[...], axis=1, keepdims=True)
    ar_ref[...] = jnp.sum(xl2 * attr_ref[...], axis=1, keepdims=True)
    s1_ref[...] = jnp.sum(x1 * att1_ref[...], axis=1, keepdims=True)
    s2_ref[...] = jnp.sum(x1 * att2_ref[...], axis=1, keepdims=True)


def _mid_call(xl, acc, uc, den, b, Wl, attl, attr, W1, att1, att2, blk):
    n, c1 = xl.shape
    c2 = Wl.shape[0]
    grid = n // blk
    full = lambda s: pl.BlockSpec(s, lambda i: tuple(0 for _ in s))
    return pl.pallas_call(
        _mid_body,
        grid=(grid,),
        in_specs=[
            pl.BlockSpec((blk, c1), lambda i: (i, 0)),
            pl.BlockSpec((2, blk, c1), lambda i: (0, i, 0)),
            pl.BlockSpec((2, blk, 1), lambda i: (0, i, 0)),
            pl.BlockSpec((2, blk, 1), lambda i: (0, i, 0)),
            full((1, c1)),
            full((c2, c1)), full((1, c2)), full((1, c2)),
            full((c2, c1)), full((1, c2)), full((1, c2)),
        ],
        out_specs=[
            pl.BlockSpec((blk, c2), lambda i: (i, 0)),
            pl.BlockSpec((blk, 1), lambda i: (i, 0)),
            pl.BlockSpec((blk, 1), lambda i: (i, 0)),
            pl.BlockSpec((blk, 1), lambda i: (i, 0)),
            pl.BlockSpec((blk, 1), lambda i: (i, 0)),
        ],
        out_shape=[
            jax.ShapeDtypeStruct((n, c2), jnp.float32),
            jax.ShapeDtypeStruct((n, 1), jnp.float32),
            jax.ShapeDtypeStruct((n, 1), jnp.float32),
            jax.ShapeDtypeStruct((n, 1), jnp.float32),
            jax.ShapeDtypeStruct((n, 1), jnp.float32),
        ],
    )(xl, acc, uc, den, b, Wl, attl, attr, W1, att1, att2)


def _final_body(xl_ref, acc_ref, uc_ref, den_ref, b_ref, out_ref):
    xl = xl_ref[...]
    total = acc_ref[0] + acc_ref[1] + xl * (uc_ref[0] + uc_ref[1])
    z = total / (den_ref[0] + den_ref[1] + EPS) + b_ref[...]
    m = jnp.max(z, axis=1, keepdims=True)
    e = jnp.exp(z - m)
    out_ref[...] = z - m - jnp.log(jnp.sum(e, axis=1, keepdims=True))


def _final_call(xl, acc, uc, den, b, blk):
    n, c = xl.shape
    grid = n // blk
    return pl.pallas_call(
        _final_body,
        grid=(grid,),
        in_specs=[
            pl.BlockSpec((blk, c), lambda i: (i, 0)),
            pl.BlockSpec((2, blk, c), lambda i: (0, i, 0)),
            pl.BlockSpec((2, blk, 1), lambda i: (0, i, 0)),
            pl.BlockSpec((2, blk, 1), lambda i: (0, i, 0)),
            pl.BlockSpec((1, c), lambda i: (0, 0)),
        ],
        out_specs=pl.BlockSpec((blk, c), lambda i: (i, 0)),
        out_shape=jax.ShapeDtypeStruct((n, c), jnp.float32),
    )(xl, acc, uc, den, b)


def _sc_layer_body(n, n_acc, c, eb, n_chunks, e_tot,
                   src_r, dst_r, al_r, ar_r, s1_r, s2_r, xl_r,
                   sig_r, den_r, uc_r, acc_r,
                   sidx0, sidx1, didx0, didx1, dsc0, dsc1,
                   al0, al1, ar0, ar1, s10, s11_, s20, s21_,
                   ex0, ex1, u0, u1, w0, w1, sg0, sg1,
                   rows0, rows1, zb_v,
                   den_sh, uc_sh, acc_sh,
                   sem_g0, sem_g1, sem_i0, sem_i1, sem_s0, sem_s1,
                   sem_a0, sem_a1):
    cid = lax.axis_index("c")
    sid = lax.axis_index("s")
    wid = sid * NC + cid
    base = wid * eb
    rpt = n_acc // NS
    row0 = pl.multiple_of(sid * rpt, K)

    zeros16 = jnp.zeros((LANES,), jnp.float32)

    @pl.loop(0, rpt // LANES)
    def _zs(i):
        zb_v[pl.ds(i * LANES, LANES)] = zeros16

    @pl.loop(0, K)
    def _zr(r):
        for g in range(c // LANES):
            rows0[r, pl.ds(g * LANES, LANES)] = zeros16

    pltpu.sync_copy(zb_v, den_sh.at[pl.ds(row0, rpt)])
    pltpu.sync_copy(zb_v, uc_sh.at[pl.ds(row0, rpt)])
    for k in range(rpt // K):
        pltpu.sync_copy(rows0, acc_sh.at[pl.ds(row0 + k * K, K)])

    plsc.subcore_barrier()

    sidx = (sidx0, sidx1)
    didx = (didx0, didx1)
    dsc = (dsc0, dsc1)
    alb = (al0, al1)
    arb = (ar0, ar1)
    s1b = (s10, s11_)
    s2b = (s20, s21_)
    exb = (ex0, ex1)
    ub = (u0, u1)
    wb = (w0, w1)
    sgb = (sg0, sg1)
    rows = (rows0, rows1)
    sem_g = (sem_g0, sem_g1)
    sem_i = (sem_i0, sem_i1)
    sem_s = (sem_s0, sem_s1)
    sem_a = (sem_a0, sem_a1)

    def drain_adds(b):
        pltpu.make_async_copy(exb[b], den_sh.at[pl.ds(0, K)], sem_a[b]).wait()
        pltpu.make_async_copy(ub[b], uc_sh.at[pl.ds(0, K)], sem_a[b]).wait()
        pltpu.make_async_copy(rows[b], acc_sh.at[pl.ds(0, K)], sem_a[b]).wait()

    def echunk(ci):
        cw = lax.rem(ci, n_chunks)
        return pl.multiple_of(base + cw * K, K)

    def fire_idx(ci, b):
        o = echunk(ci)
        pltpu.async_copy(src_r.at[pl.ds(o, K)], sidx[b], sem_i[b])
        pltpu.async_copy(dst_r.at[pl.ds(o, K)], didx[b], sem_i[b])

    def wait_idx(b):
        pltpu.make_async_copy(src_r.at[pl.ds(0, K)], sidx[b], sem_i[b]).wait()
        pltpu.make_async_copy(dst_r.at[pl.ds(0, K)], didx[b], sem_i[b]).wait()

    def fire_gathers(b):
        pltpu.async_copy(al_r.at[sidx[b]], alb[b], sem_g[b])
        pltpu.async_copy(ar_r.at[didx[b]], arb[b], sem_g[b])
        pltpu.async_copy(s1_r.at[sidx[b]], s1b[b], sem_g[b])
        pltpu.async_copy(s2_r.at[didx[b]], s2b[b], sem_g[b])
        pltpu.async_copy(xl_r.at[sidx[b]], rows[b], sem_g[b])

    def wait_gathers(b):
        pltpu.make_async_copy(al_r.at[sidx[b]], alb[b], sem_g[b]).wait()
        pltpu.make_async_copy(ar_r.at[didx[b]], arb[b], sem_g[b]).wait()
        pltpu.make_async_copy(s1_r.at[sidx[b]], s1b[b], sem_g[b]).wait()
        pltpu.make_async_copy(s2_r.at[didx[b]], s2b[b], sem_g[b]).wait()
        pltpu.make_async_copy(xl_r.at[sidx[b]], rows[b], sem_g[b]).wait()

    def compute(ci, b):
        o = echunk(ci)
        for j in range(K // LANES):
            sl = pl.ds(j * LANES, LANES)
            a = alb[b][sl] + arb[b][sl]
            a = jnp.maximum(a, a * NEG)
            ex = jnp.exp(a)
            t = s1b[b][sl] + s2b[b][sl]
            t = jnp.maximum(t, t * NEG)
            sg = 1.0 / (1.0 + jnp.exp(-t))
            gid = o + j * LANES + lax.iota(jnp.int32, LANES)
            ex = ex * (gid < e_tot).astype(jnp.float32)
            w16 = ex * sg
            sgb[b][sl] = sg
            exb[b][sl] = ex
            ub[b][sl] = ex - w16
            wb[b][sl] = w16
            dsc[b][sl] = didx[b][sl]

    def scale(b):
        @pl.loop(0, K, unroll=4)
        def _scale(e):
            bw = plsc.load_gather(wb[b], [jnp.broadcast_to(e, (LANES,))])
            for g in range(c // LANES):
                rows[b][e, pl.ds(g * LANES, LANES)] = (
                    rows[b][e, pl.ds(g * LANES, LANES)] * bw)

    pltpu.sync_copy(src_r.at[pl.ds(base, K)], sidx[0])
    pltpu.sync_copy(dst_r.at[pl.ds(base, K)], didx[0])
    fire_gathers(0)
    fire_idx(1, 1)

    @pl.loop(0, n_chunks, step=2)
    def _chunk(co):
        for b in range(2):
            ci = co + b
            nb = 1 - b
            wait_idx(nb)

            @pl.when(ci > 0)
            def _():
                pltpu.make_async_copy(
                    sgb[nb], sig_r.at[pl.ds(0, K)], sem_s[nb]).wait()
                drain_adds(nb)
            fire_gathers(nb)
            wait_gathers(b)
            compute(ci, b)
            o = echunk(ci)
            pltpu.async_copy(sgb[b], sig_r.at[pl.ds(o, K)], sem_s[b])
            pltpu.async_copy(exb[b], den_sh.at[dsc[b]], sem_a[b], add=True)
            pltpu.async_copy(ub[b], uc_sh.at[dsc[b]], sem_a[b], add=True)
            fire_idx(ci + 2, b)
            scale(b)
            pltpu.async_copy(rows[b], acc_sh.at[dsc[b]], sem_a[b], add=True)

    wait_idx(1)
    wait_gathers(0)
    pltpu.make_async_copy(sgb[1], sig_r.at[pl.ds(0, K)], sem_s[1]).wait()
    drain_adds(1)

    plsc.subcore_barrier()

    @pl.when(sid == 0)
    def _out_scal():
        cb0 = pl.multiple_of(cid * n_acc, K)
        pltpu.sync_copy(den_sh, den_r.at[pl.ds(cb0, n_acc)])
        pltpu.sync_copy(uc_sh, uc_r.at[pl.ds(cb0, n_acc)])

    ob = pl.multiple_of(cid * n_acc + row0, K)
    for k in range(rpt // K):
        pltpu.sync_copy(acc_sh.at[pl.ds(row0 + k * K, K)],
                        acc_r.at[pl.ds(ob + k * K, K)])


def _sc_layer(src, dst, al, ar, s1, s2, xl, e_tot, ep):
    n, c = xl.shape
    n_acc = ((n + NS * K - 1) // (NS * K)) * (NS * K)
    eb = ep // NW
    n_chunks = eb // K
    body = functools.partial(_sc_layer_body, n, n_acc, c, eb, n_chunks, e_tot)
    mesh = plsc.VectorSubcoreMesh(core_axis_name="c", subcore_axis_name="s",
                                  num_cores=NC, num_subcores=NS)
    f32, i32 = jnp.float32, jnp.int32
    fn = pl.kernel(
        body,
        out_type=[
            jax.ShapeDtypeStruct((ep,), f32),
            jax.ShapeDtypeStruct((NC * n_acc,), f32),
            jax.ShapeDtypeStruct((NC * n_acc,), f32),
            jax.ShapeDtypeStruct((NC * n_acc, c), f32),
        ],
        mesh=mesh,
        scratch_types=(
            [pltpu.VMEM((K,), i32) for _ in range(6)]
            + [pltpu.VMEM((K,), f32) for _ in range(16)]
            + [pltpu.VMEM((K, c), f32) for _ in range(2)]
            + [pltpu.VMEM((n_acc // NS,), f32)]
            + [pltpu.VMEM_SHARED((n_acc,), f32),
               pltpu.VMEM_SHARED((n_acc,), f32),
               pltpu.VMEM_SHARED((n_acc, c), f32)]
            + [pltpu.SemaphoreType.DMA for _ in range(8)]
        ),
        compiler_params=pltpu.CompilerParams(needs_layout_passes=False,
                                             use_tc_tiling_on_sc=False),
    )
    sig, den, uc, acc = fn(src, dst, al, ar, s1, s2, xl)
    den = den.reshape(NC, n_acc, 1)
    uc = uc.reshape(NC, n_acc, 1)
    acc = acc.reshape(NC, n_acc, c)
    return sig, den, uc, acc


def kernel(x, edge_index, Wl1, attl1, attr1, W11, att11, att12, b1,
           Wl2, attl2, attr2, W12, att21, att22, b2):
    n, _ = x.shape
    e = edge_index.shape[1]
    e_tot = e + n
    ep = ((e_tot + NW * 2 * K - 1) // (NW * 2 * K)) * (NW * 2 * K)
    npad = ep - e_tot
    loop = jnp.arange(n, dtype=edge_index.dtype)
    pad = jnp.arange(npad, dtype=edge_index.dtype) % n
    src = jnp.concatenate([edge_index[0], loop, pad])
    dst = jnp.concatenate([edge_index[1], loop, pad])

    blk = 400

    xl1, al1, ar1, s11, s21 = _dense_call(
        x, Wl1, attl1.reshape(1, -1), attr1.reshape(1, -1),
        W11, att11.reshape(1, -1), att12.reshape(1, -1), blk)

    sig1p, den1, uc1, acc1 = _sc_layer(
        src, dst, al1.reshape(-1), ar1.reshape(-1),
        s11.reshape(-1), s21.reshape(-1), xl1, e_tot, ep)

    xl2, al2, ar2, s12, s22 = _mid_call(
        xl1, acc1, uc1, den1,
        b1.reshape(1, -1), Wl2, attl2.reshape(1, -1), attr2.reshape(1, -1),
        W12, att21.reshape(1, -1), att22.reshape(1, -1), blk)

    sig2p, den2, uc2, acc2 = _sc_layer(
        src, dst, al2.reshape(-1), ar2.reshape(-1),
        s12.reshape(-1), s22.reshape(-1), xl2, e_tot, ep)

    out = _final_call(xl2, acc2, uc2, den2, b2.reshape(1, -1), blk)

    return out, sig1p[:e_tot], sig2p[:e_tot]


# speedup vs baseline: 1.6104x; 1.0179x over previous
import functools

import jax
import jax.numpy as jnp
from jax import lax
from jax.experimental import pallas as pl
from jax.experimental.pallas import tpu as pltpu
from jax.experimental.pallas import tpu_sc as plsc

NEG = 0.2
EPS = 1e-16
NC, NS, LANES = 2, 16, 16
NW = NC * NS
K = 128


def _dense_body(x_ref, wl_ref, attl_ref, attr_ref, w1_ref, att1_ref, att2_ref,
                xl_ref, al_ref, ar_ref, s1_ref, s2_ref):
    xb = x_ref[...]
    xl = jnp.dot(xb, wl_ref[...].T, preferred_element_type=jnp.float32)
    xl_ref[...] = xl
    x1 = jnp.dot(xb, w1_ref[...].T, preferred_element_type=jnp.float32)
    al_ref[...] = jnp.sum(xl * attl_ref[...], axis=1, keepdims=True)
    ar_ref[...] = jnp.sum(xl * attr_ref[...], axis=1, keepdims=True)
    s1_ref[...] = jnp.sum(x1 * att1_ref[...], axis=1, keepdims=True)
    s2_ref[...] = jnp.sum(x1 * att2_ref[...], axis=1, keepdims=True)


def _dense_call(x, Wl, attl, attr, W1, att1, att2, blk):
    n, f = x.shape
    c = Wl.shape[0]
    grid = n // blk
    full = lambda s: pl.BlockSpec(s, lambda i: (0, 0))
    return pl.pallas_call(
        _dense_body,
        grid=(grid,),
        in_specs=[
            pl.BlockSpec((blk, f), lambda i: (i, 0)),
            full((c, f)), full((1, c)), full((1, c)),
            full((c, f)), full((1, c)), full((1, c)),
        ],
        out_specs=[
            pl.BlockSpec((blk, c), lambda i: (i, 0)),
            pl.BlockSpec((blk, 1), lambda i: (i, 0)),
            pl.BlockSpec((blk, 1), lambda i: (i, 0)),
            pl.BlockSpec((blk, 1), lambda i: (i, 0)),
            pl.BlockSpec((blk, 1), lambda i: (i, 0)),
        ],
        out_shape=[
            jax.ShapeDtypeStruct((n, c), jnp.float32),
            jax.ShapeDtypeStruct((n, 1), jnp.float32),
            jax.ShapeDtypeStruct((n, 1), jnp.float32),
            jax.ShapeDtypeStruct((n, 1), jnp.float32),
            jax.ShapeDtypeStruct((n, 1), jnp.float32),
        ],
    )(x, Wl, attl, attr, W1, att1, att2)


def _mid_body(xl_ref, acc_ref, uc_ref, den_ref, b_ref,
              wl_ref, attl_ref, attr_ref, w1_ref, att1_ref, att2_ref,
              xl2_ref, al_ref, ar_ref, s1_ref, s2_ref):
    xl = xl_ref[...]
    total = acc_ref[0] + acc_ref[1] + xl * (uc_ref[0] + uc_ref[1])
    h = total / (den_ref[0] + den_ref[1] + EPS) + b_ref[...]
    h = jnp.maximum(h, 0.0)
    xl2 = jnp.dot(h, wl_ref[...].T, preferred_element_type=jnp.float32)
    xl2_ref[...] = xl2
    x1 = jnp.dot(h, w1_ref[...].T, preferred_element_type=jnp.float32)
    al_ref[...] = jnp.sum(xl2 * attl_ref[...], axis=1, keepdims=True)
    ar_ref[...] = jnp.sum(xl2 * attr_ref[...], axis=1, keepdims=True)
    s1_ref[...] = jnp.sum(x1 * att1_ref[...], axis=1, keepdims=True)
    s2_ref[...] = jnp.sum(x1 * att2_ref[...], axis=1, keepdims=True)


def _mid_call(xl, acc, uc, den, b, Wl, attl, attr, W1, att1, att2, blk):
    n, c1 = xl.shape
    c2 = Wl.shape[0]
    grid = n // blk
    full = lambda s: pl.BlockSpec(s, lambda i: tuple(0 for _ in s))
    return pl.pallas_call(
        _mid_body,
        grid=(grid,),
        in_specs=[
            pl.BlockSpec((blk, c1), lambda i: (i, 0)),
            pl.BlockSpec((2, blk, c1), lambda i: (0, i, 0)),
            pl.BlockSpec((2, blk, 1), lambda i: (0, i, 0)),
            pl.BlockSpec((2, blk, 1), lambda i: (0, i, 0)),
            full((1, c1)),
            full((c2, c1)), full((1, c2)), full((1, c2)),
            full((c2, c1)), full((1, c2)), full((1, c2)),
        ],
        out_specs=[
            pl.BlockSpec((blk, c2), lambda i: (i, 0)),
            pl.BlockSpec((blk, 1), lambda i: (i, 0)),
            pl.BlockSpec((blk, 1), lambda i: (i, 0)),
            pl.BlockSpec((blk, 1), lambda i: (i, 0)),
            pl.BlockSpec((blk, 1), lambda i: (i, 0)),
        ],
        out_shape=[
            jax.ShapeDtypeStruct((n, c2), jnp.float32),
            jax.ShapeDtypeStruct((n, 1), jnp.float32),
            jax.ShapeDtypeStruct((n, 1), jnp.float32),
            jax.ShapeDtypeStruct((n, 1), jnp.float32),
            jax.ShapeDtypeStruct((n, 1), jnp.float32),
        ],
    )(xl, acc, uc, den, b, Wl, attl, attr, W1, att1, att2)


def _final_body(xl_ref, acc_ref, uc_ref, den_ref, b_ref, out_ref):
    xl = xl_ref[...]
    total = acc_ref[0] + acc_ref[1] + xl * (uc_ref[0] + uc_ref[1])
    z = total / (den_ref[0] + den_ref[1] + EPS) + b_ref[...]
    m = jnp.max(z, axis=1, keepdims=True)
    e = jnp.exp(z - m)
    out_ref[...] = z - m - jnp.log(jnp.sum(e, axis=1, keepdims=True))


def _final_call(xl, acc, uc, den, b, blk):
    n, c = xl.shape
    grid = n // blk
    return pl.pallas_call(
        _final_body,
        grid=(grid,),
        in_specs=[
            pl.BlockSpec((blk, c), lambda i: (i, 0)),
            pl.BlockSpec((2, blk, c), lambda i: (0, i, 0)),
            pl.BlockSpec((2, blk, 1), lambda i: (0, i, 0)),
            pl.BlockSpec((2, blk, 1), lambda i: (0, i, 0)),
            pl.BlockSpec((1, c), lambda i: (0, 0)),
        ],
        out_specs=pl.BlockSpec((blk, c), lambda i: (i, 0)),
        out_shape=jax.ShapeDtypeStruct((n, c), jnp.float32),
    )(xl, acc, uc, den, b)


def _sc_layer_body(n, n_acc, c, eb, n_chunks, e_tot,
                   src_r, dst_r, al_r, ar_r, s1_r, s2_r, xl_r,
                   sig_r, den_r, uc_r, acc_r,
                   sidx0, sidx1, didx0, didx1, dsc0, dsc1,
                   al0, al1, ar0, ar1, s10, s11_, s20, s21_,
                   ex0, ex1, u0, u1, w0, w1, sg0, sg1,
                   rows0, rows1, zb_v,
                   den_sh, uc_sh, acc_sh,
                   sem_g0, sem_g1, sem_i0, sem_i1, sem_s0, sem_s1,
                   sem_a0, sem_a1):
    cid = lax.axis_index("c")
    sid = lax.axis_index("s")
    wid = sid * NC + cid
    base = wid * eb
    rpt = n_acc // NS
    row0 = pl.multiple_of(sid * rpt, K)

    zeros16 = jnp.zeros((LANES,), jnp.float32)

    @pl.loop(0, rpt // LANES)
    def _zs(i):
        zb_v[pl.ds(i * LANES, LANES)] = zeros16

    @pl.loop(0, K)
    def _zr(r):
        for g in range(c // LANES):
            rows0[r, pl.ds(g * LANES, LANES)] = zeros16

    pltpu.sync_copy(zb_v, den_sh.at[pl.ds(row0, rpt)])
    pltpu.sync_copy(zb_v, uc_sh.at[pl.ds(row0, rpt)])
    for k in range(rpt // K):
        pltpu.sync_copy(rows0, acc_sh.at[pl.ds(row0 + k * K, K)])

    plsc.subcore_barrier()

    sidx = (sidx0, sidx1)
    didx = (didx0, didx1)
    dsc = (dsc0, dsc1)
    alb = (al0, al1)
    arb = (ar0, ar1)
    s1b = (s10, s11_)
    s2b = (s20, s21_)
    exb = (ex0, ex1)
    ub = (u0, u1)
    wb = (w0, w1)
    sgb = (sg0, sg1)
    rows = (rows0, rows1)
    sem_g = (sem_g0, sem_g1)
    sem_i = (sem_i0, sem_i1)
    sem_s = (sem_s0, sem_s1)
    sem_a = (sem_a0, sem_a1)

    def drain_adds(b):
        pltpu.make_async_copy(exb[b], den_sh.at[pl.ds(0, K)], sem_a[b]).wait()
        pltpu.make_async_copy(ub[b], uc_sh.at[pl.ds(0, K)], sem_a[b]).wait()
        pltpu.make_async_copy(rows[b], acc_sh.at[pl.ds(0, K)], sem_a[b]).wait()

    def echunk(ci):
        cw = lax.rem(ci, n_chunks)
        return pl.multiple_of(base + cw * K, K)

    def fire_idx(ci, b):
        o = echunk(ci)
        pltpu.async_copy(src_r.at[pl.ds(o, K)], sidx[b], sem_i[b])
        pltpu.async_copy(dst_r.at[pl.ds(o, K)], didx[b], sem_i[b])

    def wait_idx(b):
        pltpu.make_async_copy(src_r.at[pl.ds(0, K)], sidx[b], sem_i[b]).wait()
        pltpu.make_async_copy(dst_r.at[pl.ds(0, K)], didx[b], sem_i[b]).wait()

    def fire_gathers(b):
        pltpu.async_copy(al_r.at[sidx[b]], alb[b], sem_g[b])
        pltpu.async_copy(ar_r.at[didx[b]], arb[b], sem_g[b])
        pltpu.async_copy(s1_r.at[sidx[b]], s1b[b], sem_g[b])
        pltpu.async_copy(s2_r.at[didx[b]], s2b[b], sem_g[b])
        pltpu.async_copy(xl_r.at[sidx[b]], rows[b], sem_g[b])

    def wait_gathers(b):
        pltpu.make_async_copy(al_r.at[sidx[b]], alb[b], sem_g[b]).wait()
        pltpu.make_async_copy(ar_r.at[didx[b]], arb[b], sem_g[b]).wait()
        pltpu.make_async_copy(s1_r.at[sidx[b]], s1b[b], sem_g[b]).wait()
        pltpu.make_async_copy(s2_r.at[didx[b]], s2b[b], sem_g[b]).wait()
        pltpu.make_async_copy(xl_r.at[sidx[b]], rows[b], sem_g[b]).wait()

    def compute(ci, b):
        o = echunk(ci)
        for j in range(K // LANES):
            sl = pl.ds(j * LANES, LANES)
            a = alb[b][sl] + arb[b][sl]
            a = jnp.maximum(a, a * NEG)
            ex = jnp.exp(a)
            t = s1b[b][sl] + s2b[b][sl]
            t = jnp.maximum(t, t * NEG)
            sg = 1.0 / (1.0 + jnp.exp(-t))
            gid = o + j * LANES + lax.iota(jnp.int32, LANES)
            ex = ex * (gid < e_tot).astype(jnp.float32)
            w16 = ex * sg
            sgb[b][sl] = sg
            exb[b][sl] = ex
            ub[b][sl] = ex - w16
            wb[b][sl] = w16
            dsc[b][sl] = didx[b][sl]

    def scale(b):
        @pl.loop(0, K, unroll=4)
        def _scale(e):
            bw = plsc.load_gather(wb[b], [jnp.broadcast_to(e, (LANES,))])
            for g in range(c // LANES):
                rows[b][e, pl.ds(g * LANES, LANES)] = (
                    rows[b][e, pl.ds(g * LANES, LANES)] * bw)

    pltpu.sync_copy(src_r.at[pl.ds(base, K)], sidx[0])
    pltpu.sync_copy(dst_r.at[pl.ds(base, K)], didx[0])
    fire_gathers(0)
    fire_idx(1, 1)

    @pl.loop(0, n_chunks, step=2)
    def _chunk(co):
        for b in range(2):
            ci = co + b
            nb = 1 - b
            wait_idx(nb)

            @pl.when(ci > 0)
            def _():
                pltpu.make_async_copy(
                    sgb[nb], sig_r.at[pl.ds(0, K)], sem_s[nb]).wait()
                drain_adds(nb)
            fire_gathers(nb)
            wait_gathers(b)
            compute(ci, b)
            o = echunk(ci)
            pltpu.async_copy(sgb[b], sig_r.at[pl.ds(o, K)], sem_s[b])
            pltpu.async_copy(exb[b], den_sh.at[dsc[b]], sem_a[b], add=True)
            pltpu.async_copy(ub[b], uc_sh.at[dsc[b]], sem_a[b], add=True)
            fire_idx(ci + 2, b)
            scale(b)
            pltpu.async_copy(rows[b], acc_sh.at[dsc[b]], sem_a[b], add=True)

    wait_idx(1)
    wait_gathers(0)
    pltpu.make_async_copy(sgb[1], sig_r.at[pl.ds(0, K)], sem_s[1]).wait()
    drain_adds(1)

    plsc.subcore_barrier()

    @pl.when(sid == 0)
    def _out_scal():
        cb0 = pl.multiple_of(cid * n_acc, K)
        pltpu.sync_copy(den_sh, den_r.at[pl.ds(cb0, n_acc)])
        pltpu.sync_copy(uc_sh, uc_r.at[pl.ds(cb0, n_acc)])

    ob = pl.multiple_of(cid * n_acc + row0, K)
    for k in range(rpt // K):
        pltpu.sync_copy(acc_sh.at[pl.ds(row0 + k * K, K)],
                        acc_r.at[pl.ds(ob + k * K, K)])


def _sc_layer(src, dst, al, ar, s1, s2, xl, e_tot, ep):
    n, c = xl.shape
    n_acc = ((n + NS * K - 1) // (NS * K)) * (NS * K)
    eb = ep // NW
    n_chunks = eb // K
    body = functools.partial(_sc_layer_body, n, n_acc, c, eb, n_chunks, e_tot)
    mesh = plsc.VectorSubcoreMesh(core_axis_name="c", subcore_axis_name="s",
                                  num_cores=NC, num_subcores=NS)
    f32, i32 = jnp.float32, jnp.int32
    fn = pl.kernel(
        body,
        out_type=[
            jax.ShapeDtypeStruct((ep,), f32),
            jax.ShapeDtypeStruct((NC * n_acc,), f32),
            jax.ShapeDtypeStruct((NC * n_acc,), f32),
            jax.ShapeDtypeStruct((NC * n_acc, c), f32),
        ],
        mesh=mesh,
        scratch_types=(
            [pltpu.VMEM((K,), i32) for _ in range(6)]
            + [pltpu.VMEM((K,), f32) for _ in range(16)]
            + [pltpu.VMEM((K, c), f32) for _ in range(2)]
            + [pltpu.VMEM((n_acc // NS,), f32)]
            + [pltpu.VMEM_SHARED((n_acc,), f32),
               pltpu.VMEM_SHARED((n_acc,), f32),
               pltpu.VMEM_SHARED((n_acc, c), f32)]
            + [pltpu.SemaphoreType.DMA for _ in range(8)]
        ),
        compiler_params=pltpu.CompilerParams(needs_layout_passes=False,
                                             use_tc_tiling_on_sc=False),
    )
    sig, den, uc, acc = fn(src, dst, al, ar, s1, s2, xl)
    den = den.reshape(NC, n_acc)[:, :n]
    uc = uc.reshape(NC, n_acc)[:, :n]
    acc = acc.reshape(NC, n_acc, c)[:, :n, :]
    return sig, den, uc, acc


def kernel(x, edge_index, Wl1, attl1, attr1, W11, att11, att12, b1,
           Wl2, attl2, attr2, W12, att21, att22, b2):
    n, _ = x.shape
    e = edge_index.shape[1]
    e_tot = e + n
    ep = ((e_tot + NW * 2 * K - 1) // (NW * 2 * K)) * (NW * 2 * K)
    npad = ep - e_tot
    loop = jnp.arange(n, dtype=edge_index.dtype)
    pad = jnp.arange(npad, dtype=edge_index.dtype) % n
    src = jnp.concatenate([edge_index[0], loop, pad])
    dst = jnp.concatenate([edge_index[1], loop, pad])

    blk = 400

    xl1, al1, ar1, s11, s21 = _dense_call(
        x, Wl1, attl1.reshape(1, -1), attr1.reshape(1, -1),
        W11, att11.reshape(1, -1), att12.reshape(1, -1), blk)

    sig1p, den1, uc1, acc1 = _sc_layer(
        src, dst, al1.reshape(-1), ar1.reshape(-1),
        s11.reshape(-1), s21.reshape(-1), xl1, e_tot, ep)

    xl2, al2, ar2, s12, s22 = _mid_call(
        xl1, acc1, uc1.reshape(NC, n, 1), den1.reshape(NC, n, 1),
        b1.reshape(1, -1), Wl2, attl2.reshape(1, -1), attr2.reshape(1, -1),
        W12, att21.reshape(1, -1), att22.reshape(1, -1), blk)

    sig2p, den2, uc2, acc2 = _sc_layer(
        src, dst, al2.reshape(-1), ar2.reshape(-1),
        s12.reshape(-1), s22.reshape(-1), xl2, e_tot, ep)

    out = _final_call(xl2, acc2, uc2.reshape(NC, n, 1), den2.reshape(NC, n, 1),
                      b2.reshape(1, -1), blk)

    return out, sig1p[:e_tot], sig2p[:e_tot]
